# baseline (speedup 1.0000x reference)
import jax
import jax.numpy as jnp
from jax import lax
from jax.experimental import pallas as pl
from jax.experimental.pallas import tpu as pltpu
from jax.experimental.pallas import tpu_sc as plsc

N = 10000
E = 320000
D = 128
HD = 128
F = HD + D
FP = F // 2
_NC = 2
_NS = 16
_NW = _NC * _NS
_EPW = E // _NW
_BG = 80
_NBG = _EPW // _BG
_BS = 80
_NBS = _EPW // _BS
_NPAD = 10240
_RPS = _NPAD // _NS

_f32 = jnp.float32
_bf16 = jnp.bfloat16

def _mesh():
    return plsc.VectorSubcoreMesh(
        core_axis_name="c", subcore_axis_name="s",
        num_cores=_NC, num_subcores=_NS)


def _pack_rows(a_f32):
    m = a_f32.shape[0]
    return pltpu.bitcast(a_f32.astype(_bf16).reshape(2 * m, FP), jnp.int32)

def _node1_body(x_ref, bw_ref, bb_ref, w1_ref, b1_ref, w2_ref, b2_ref,
                cw_ref, cb_ref, x_out_ref, p_ref, q_ref):
    x = x_ref[...]
    mu = jnp.mean(x, axis=0, keepdims=True)
    xc = x - mu
    var = jnp.mean(xc * xc, axis=0, keepdims=True)
    xb = xc * lax.rsqrt(var + 1e-5) * bw_ref[...] + bb_ref[...]
    x_out_ref[...] = xb
    t = jnp.maximum(
        jnp.dot(xb, w1_ref[...], preferred_element_type=_f32) + b1_ref[...],
        0.0)
    h0 = jnp.tanh(
        jnp.dot(t, w2_ref[...], preferred_element_type=_f32) + b2_ref[...])
    feat = jnp.concatenate([h0, xb], axis=1)
    cw = cw_ref[...]
    wb = cw[F:]
    wa = cw[:F] - wb
    p_ref[...] = _pack_rows(
        jnp.dot(feat, wa, preferred_element_type=_f32) + cb_ref[...])
    q_ref[...] = _pack_rows(jnp.dot(feat, wb, preferred_element_type=_f32))


def _conv2_body(pg_ref, qg_ref, w_ref, b_ref, m_ref):
    pb = pltpu.bitcast(pg_ref[...], _bf16)
    qb = pltpu.bitcast(qg_ref[...], _bf16)
    t = jnp.maximum(pb + qb, 0).reshape(pg_ref.shape[0], F)
    m_ref[...] = jnp.tanh(
        jnp.dot(t, w_ref[...], preferred_element_type=_f32) + b_ref[...])


def _edgered_body(rg_ref, sg_ref, w2_ref, b2_ref, o_ref):
    rb_ = pltpu.bitcast(rg_ref[...], _bf16)
    sb_ = pltpu.bitcast(sg_ref[...], _bf16)
    t = jnp.maximum(rb_ + sb_, 0).reshape(rg_ref.shape[0], F)
    z = lax.dot_general(w2_ref[...], t, (((1,), (1,)), ((), ())),
                        preferred_element_type=_f32)
    o_ref[...] = (1.0 / (1.0 + jnp.exp(-(z + b2_ref[0, 0])))).reshape(
        1, 1, z.shape[1])


def _node2_body(h1_ref, h2_ref, xb_ref, w_ref, b_ref, r_ref, s_ref):
    feat = jnp.concatenate([h1_ref[0] + h2_ref[0], xb_ref[...]], axis=1)
    w = w_ref[...]
    r_ref[...] = _pack_rows(
        jnp.dot(feat, w[:F], preferred_element_type=_f32) + b_ref[...])
    s_ref[...] = _pack_rows(jnp.dot(feat, w[F:], preferred_element_type=_f32))


def _sc_gather2(p_hbm, q_hbm, ia_hbm, ib_hbm, pg_hbm, qg_hbm,
                iaall, iball, bp0, bp1, bp2, bp3, bq0, bq1, bq2, bq3,
                sp0, sp1, sp2, sp3, sq0, sq1, sq2, sq3,
                tp0, tp1, tp2, tp3, tq0, tq1, tq2, tq3):
    wid = lax.axis_index("s") * _NC + lax.axis_index("c")
    base = wid * _EPW
    pltpu.sync_copy(ia_hbm.at[pl.ds(base, _EPW)], iaall)
    pltpu.sync_copy(ib_hbm.at[pl.ds(base, _EPW)], iball)
    bp, bq = (bp0, bp1, bp2, bp3), (bq0, bq1, bq2, bq3)
    sp, sq = (sp0, sp1, sp2, sp3), (sq0, sq1, sq2, sq3)
    tp, tq = (tp0, tp1, tp2, tp3), (tq0, tq1, tq2, tq3)

    def gather(i, b):
        pltpu.async_copy(p_hbm.at[iaall.at[pl.ds(i * _BG, _BG)]], bp[b], sp[b])
        pltpu.async_copy(q_hbm.at[iball.at[pl.ds(i * _BG, _BG)]], bq[b], sq[b])

    def wait_gather(b):
        pltpu.make_async_copy(p_hbm.at[pl.ds(0, _BG)], bp[b], sp[b]).wait()
        pltpu.make_async_copy(q_hbm.at[pl.ds(0, _BG)], bq[b], sq[b]).wait()

    def store(i, b):
        off = base + i * _BG
        pltpu.async_copy(bp[b], pg_hbm.at[pl.ds(off, _BG)], tp[b])
        pltpu.async_copy(bq[b], qg_hbm.at[pl.ds(off, _BG)], tq[b])

    def wait_store(b):
        pltpu.make_async_copy(bp[b], pg_hbm.at[pl.ds(0, _BG)], tp[b]).wait()
        pltpu.make_async_copy(bq[b], qg_hbm.at[pl.ds(0, _BG)], tq[b]).wait()

    gather(0, 0)
    gather(1, 1)

    @pl.loop(0, (_NBG - 1) // 4)
    def _j(j):
        for b in range(4):
            i = j * 4 + b
            nxt = (b + 2) % 4
            if b < 2:
                @pl.when(j > 0)
                def _():
                    wait_store(nxt)
                gather(i + 2, nxt)
            elif b == 2:
                wait_store(nxt)
                gather(i + 2, nxt)
            else:
                wait_store(nxt)

                @pl.when(j < (_NBG - 1) // 4 - 1)
                def _():
                    gather(i + 2, nxt)
            wait_gather(b)
            store(i, b)

    wait_gather(0)
    store(_NBG - 1, 0)
    wait_store(2)
    wait_store(3)
    wait_store(0)


def _sc_scatter_add(msg_hbm, col_hbm, hp_hbm,
                    cv0, cv1, mb0, mb1, zbuf, acc, sc0, sc1, sm0, sm1):
    cidx = lax.axis_index("c")
    sidx = lax.axis_index("s")
    wid = sidx * _NC + cidx

    @pl.loop(0, 128)
    def _z(r):
        for ch in range(D // 16):
            zbuf[r, pl.ds(ch * 16, 16)] = jnp.zeros((16,), _f32)

    for j in range(_RPS // 128):
        pltpu.sync_copy(zbuf, acc.at[pl.ds(sidx * _RPS + j * 128, 128)])
    plsc.subcore_barrier()

    base = wid * _EPW
    cv, mb = (cv0, cv1), (mb0, mb1)
    sc_, sm = (sc0, sc1), (sm0, sm1)

    def issue(i, b):
        off = base + i * _BS
        pltpu.async_copy(col_hbm.at[pl.ds(off, _BS)], cv[b], sc_[b])
        pltpu.async_copy(msg_hbm.at[pl.ds(off, _BS)], mb[b], sm[b])

    def drain(b):
        pltpu.make_async_copy(col_hbm.at[pl.ds(0, _BS)], cv[b], sc_[b]).wait()
        pltpu.make_async_copy(msg_hbm.at[pl.ds(0, _BS)], mb[b], sm[b]).wait()

    issue(0, 0)

    @pl.loop(0, (_NBS - 1) // 2)
    def _j(j):
        for b in range(2):
            i = j * 2 + b
            issue(i + 1, 1 - b)
            drain(b)
            pltpu.sync_copy(mb[b], acc.at[cv[b]], add=True)

    drain(0)
    pltpu.sync_copy(mb[0], acc.at[cv[0]], add=True)

    plsc.subcore_barrier()
    for j in range(_RPS // 128):
        r0 = sidx * _RPS + j * 128
        pltpu.sync_copy(acc.at[pl.ds(r0, 128)],
                        hp_hbm.at[cidx, pl.ds(r0, 128)])


def _gather_pair(p_i32, q_i32, ia, ib):
    return pl.kernel(
        _sc_gather2,
        out_type=[jax.ShapeDtypeStruct((E, FP), jnp.int32)] * 2,
        mesh=_mesh(),
        scratch_types=(
            [pltpu.VMEM((_EPW,), jnp.int32)] * 2
            + [pltpu.VMEM((_BG, FP), jnp.int32)] * 8
            + [pltpu.SemaphoreType.DMA] * 16
        ),
    )(p_i32, q_i32, ia, ib)


@jax.jit
def kernel(x, edge_index, bn_w, bn_b, in1_W, in1_b, in2_W, in2_b,
           conv1_W, conv1_b, conv2_W, conv2_b, edge1_W, edge1_b,
           edge2_W, edge2_b):
    row = edge_index[0]
    col = edge_index[1]

    nblk = 5
    rb = N // nblk
    full = lambda shape: pl.BlockSpec(shape, lambda i: (0, 0))
    X, p, q = pl.pallas_call(
        _node1_body,
        out_shape=[jax.ShapeDtypeStruct((N, D), _f32)]
        + [jax.ShapeDtypeStruct((N, FP), jnp.int32)] * 2,
    )(x, bn_w.reshape(1, D), bn_b.reshape(1, D),
      in1_W, in1_b.reshape(1, HD), in2_W, in2_b.reshape(1, HD),
      conv1_W, conv1_b.reshape(1, F))

    pg, qg = _gather_pair(p, q, col, row)

    eblk = 2560
    msg = pl.pallas_call(
        _conv2_body,
        grid=(E // eblk,),
        in_specs=[
            pl.BlockSpec((eblk, FP), lambda i: (i, 0)),
            pl.BlockSpec((eblk, FP), lambda i: (i, 0)),
            full((F, HD)), full((1, HD)),
        ],
        out_specs=pl.BlockSpec((eblk, HD), lambda i: (i, 0)),
        out_shape=jax.ShapeDtypeStruct((E, HD), _f32),
    )(pg, qg, conv2_W.astype(_bf16), conv2_b.reshape(1, HD))

    hp = pl.kernel(
        _sc_scatter_add,
        out_type=jax.ShapeDtypeStruct((_NC, _NPAD, D), _f32),
        mesh=_mesh(),
        scratch_types=(
            [pltpu.VMEM((_BS,), jnp.int32)] * 2
            + [pltpu.VMEM((_BS, D), _f32)] * 2
            + [pltpu.VMEM((128, D), _f32),
               pltpu.VMEM_SHARED((_NPAD, D), _f32)]
            + [pltpu.SemaphoreType.DMA] * 4
        ),
    )(msg, col)

    r, s = pl.pallas_call(
        _node2_body,
        grid=(nblk,),
        in_specs=[
            pl.BlockSpec((1, rb, D), lambda i: (0, i, 0)),
            pl.BlockSpec((1, rb, D), lambda i: (1, i, 0)),
            pl.BlockSpec((rb, D), lambda i: (i, 0)),
            full((2 * F, F)), full((1, F)),
        ],
        out_specs=[pl.BlockSpec((rb, FP), lambda i: (i, 0))] * 2,
        out_shape=[jax.ShapeDtypeStruct((N, FP), jnp.int32)] * 2,
    )(hp, hp, X, edge1_W, edge1_b.reshape(1, F))

    rg, sg = _gather_pair(r, s, row, col)

    dblk = 2560
    out3d = pl.pallas_call(
        _edgered_body,
        grid=(E // dblk,),
        in_specs=[
            pl.BlockSpec((dblk, FP), lambda i: (i, 0)),
            pl.BlockSpec((dblk, FP), lambda i: (i, 0)),
            full((1, F)), full((1, 1)),
        ],
        out_specs=pl.BlockSpec((1, 1, dblk), lambda i: (i, 0, 0)),
        out_shape=jax.ShapeDtypeStruct((E // dblk, 1, dblk), _f32),
    )(rg, sg, edge2_W.reshape(1, F).astype(_bf16), edge2_b.reshape(1, 1))

    return out3d.reshape(E)


# speedup vs baseline: 1.0709x; 1.0709x over previous
import jax
import jax.numpy as jnp
from jax import lax
from jax.experimental import pallas as pl
from jax.experimental.pallas import tpu as pltpu
from jax.experimental.pallas import tpu_sc as plsc

N = 10000
E = 320000
D = 128
HD = 128
F = HD + D
FP = F // 2
_NC = 2
_NS = 16
_NW = _NC * _NS
_EPW = E // _NW
_BG = 80
_NBG = _EPW // _BG
_BS = 80
_NBS = _EPW // _BS
_NPAD = 10240
_RPS = _NPAD // _NS

_f32 = jnp.float32
_bf16 = jnp.bfloat16

def _mesh():
    return plsc.VectorSubcoreMesh(
        core_axis_name="c", subcore_axis_name="s",
        num_cores=_NC, num_subcores=_NS)


def _pack_rows(a_f32):
    m = a_f32.shape[0]
    return pltpu.bitcast(a_f32.astype(_bf16).reshape(2 * m, FP), jnp.int32)

def _node1_body(x_ref, bw_ref, bb_ref, w1_ref, b1_ref, w2_ref, b2_ref,
                cw_ref, cb_ref, x_out_ref, p_ref, q_ref):
    x = x_ref[...]
    mu = jnp.mean(x, axis=0, keepdims=True)
    xc = x - mu
    var = jnp.mean(xc * xc, axis=0, keepdims=True)
    xb = xc * lax.rsqrt(var + 1e-5) * bw_ref[...] + bb_ref[...]
    x_out_ref[...] = xb
    t = jnp.maximum(
        jnp.dot(xb, w1_ref[...], preferred_element_type=_f32) + b1_ref[...],
        0.0)
    h0 = jnp.tanh(
        jnp.dot(t, w2_ref[...], preferred_element_type=_f32) + b2_ref[...])
    feat = jnp.concatenate([h0, xb], axis=1)
    cw = cw_ref[...]
    wb = cw[F:]
    wa = cw[:F] - wb
    p_ref[...] = _pack_rows(
        jnp.dot(feat, wa, preferred_element_type=_f32) + cb_ref[...])
    q_ref[...] = _pack_rows(jnp.dot(feat, wb, preferred_element_type=_f32))


def _conv2_body(pg_ref, qg_ref, w_ref, b_ref, m_ref):
    pb = pltpu.bitcast(pg_ref[...], _bf16)
    qb = pltpu.bitcast(qg_ref[...], _bf16)
    t = jnp.maximum(pb + qb, 0).reshape(pg_ref.shape[0], F)
    m_ref[...] = jnp.tanh(
        jnp.dot(t, w_ref[...], preferred_element_type=_f32) + b_ref[...])


def _edgered_body(rg_ref, sg_ref, w2_ref, b2_ref, o_ref):
    rb_ = pltpu.bitcast(rg_ref[...], _bf16)
    sb_ = pltpu.bitcast(sg_ref[...], _bf16)
    t = jnp.maximum(rb_ + sb_, 0).reshape(rg_ref.shape[0], F)
    z = lax.dot_general(w2_ref[...], t, (((1,), (1,)), ((), ())),
                        preferred_element_type=_f32)
    o_ref[...] = (1.0 / (1.0 + jnp.exp(-(z + b2_ref[0, 0])))).reshape(
        1, 1, z.shape[1])


def _node2_body(h1_ref, h2_ref, xb_ref, w_ref, b_ref, r_ref, s_ref):
    feat = jnp.concatenate([h1_ref[0] + h2_ref[0], xb_ref[...]], axis=1)
    w = w_ref[...]
    r_ref[...] = _pack_rows(
        jnp.dot(feat, w[:F], preferred_element_type=_f32) + b_ref[...])
    s_ref[...] = _pack_rows(jnp.dot(feat, w[F:], preferred_element_type=_f32))


def _sc_gather2(p_hbm, q_hbm, ia_hbm, ib_hbm, pg_hbm, qg_hbm,
                iaall, iball, bp0, bp1, bp2, bp3, bq0, bq1, bq2, bq3,
                sp0, sp1, sp2, sp3, sq0, sq1, sq2, sq3,
                tp0, tp1, tp2, tp3, tq0, tq1, tq2, tq3):
    wid = lax.axis_index("s") * _NC + lax.axis_index("c")
    base = wid * _EPW
    pltpu.sync_copy(ia_hbm.at[pl.ds(base, _EPW)], iaall)
    pltpu.sync_copy(ib_hbm.at[pl.ds(base, _EPW)], iball)
    bp, bq = (bp0, bp1, bp2, bp3), (bq0, bq1, bq2, bq3)
    sp, sq = (sp0, sp1, sp2, sp3), (sq0, sq1, sq2, sq3)
    tp, tq = (tp0, tp1, tp2, tp3), (tq0, tq1, tq2, tq3)

    def gather(i, b):
        pltpu.async_copy(p_hbm.at[iaall.at[pl.ds(i * _BG, _BG)]], bp[b], sp[b])
        pltpu.async_copy(q_hbm.at[iball.at[pl.ds(i * _BG, _BG)]], bq[b], sq[b])

    def wait_gather(b):
        pltpu.make_async_copy(p_hbm.at[pl.ds(0, _BG)], bp[b], sp[b]).wait()
        pltpu.make_async_copy(q_hbm.at[pl.ds(0, _BG)], bq[b], sq[b]).wait()

    def store(i, b):
        off = base + i * _BG
        pltpu.async_copy(bp[b], pg_hbm.at[pl.ds(off, _BG)], tp[b])
        pltpu.async_copy(bq[b], qg_hbm.at[pl.ds(off, _BG)], tq[b])

    def wait_store(b):
        pltpu.make_async_copy(bp[b], pg_hbm.at[pl.ds(0, _BG)], tp[b]).wait()
        pltpu.make_async_copy(bq[b], qg_hbm.at[pl.ds(0, _BG)], tq[b]).wait()

    gather(0, 0)
    gather(1, 1)

    @pl.loop(0, (_NBG - 1) // 4)
    def _j(j):
        for b in range(4):
            i = j * 4 + b
            nxt = (b + 2) % 4
            if b < 2:
                @pl.when(j > 0)
                def _():
                    wait_store(nxt)
                gather(i + 2, nxt)
            elif b == 2:
                wait_store(nxt)
                gather(i + 2, nxt)
            else:
                wait_store(nxt)

                @pl.when(j < (_NBG - 1) // 4 - 1)
                def _():
                    gather(i + 2, nxt)
            wait_gather(b)
            store(i, b)

    wait_gather(0)
    store(_NBG - 1, 0)
    wait_store(2)
    wait_store(3)
    wait_store(0)


def _sc_scatter_add(msg_hbm, col_hbm, hp_hbm,
                    cv0, cv1, mb0, mb1, zbuf, acc, sc0, sc1, sm0, sm1):
    cidx = lax.axis_index("c")
    sidx = lax.axis_index("s")
    wid = sidx * _NC + cidx

    @pl.loop(0, 128)
    def _z(r):
        for ch in range(D // 16):
            zbuf[r, pl.ds(ch * 16, 16)] = jnp.zeros((16,), _f32)

    for j in range(_RPS // 128):
        pltpu.sync_copy(zbuf, acc.at[pl.ds(sidx * _RPS + j * 128, 128)])
    plsc.subcore_barrier()

    base = wid * _EPW
    cv, mb = (cv0, cv1), (mb0, mb1)
    sc_, sm = (sc0, sc1), (sm0, sm1)

    def issue(i, b):
        off = base + i * _BS
        pltpu.async_copy(col_hbm.at[pl.ds(off, _BS)], cv[b], sc_[b])
        pltpu.async_copy(msg_hbm.at[pl.ds(off, _BS)], mb[b], sm[b])

    def drain(b):
        pltpu.make_async_copy(col_hbm.at[pl.ds(0, _BS)], cv[b], sc_[b]).wait()
        pltpu.make_async_copy(msg_hbm.at[pl.ds(0, _BS)], mb[b], sm[b]).wait()

    issue(0, 0)

    @pl.loop(0, (_NBS - 1) // 2)
    def _j(j):
        for b in range(2):
            i = j * 2 + b
            issue(i + 1, 1 - b)
            drain(b)
            pltpu.sync_copy(mb[b], acc.at[cv[b]], add=True)

    drain(0)
    pltpu.sync_copy(mb[0], acc.at[cv[0]], add=True)

    plsc.subcore_barrier()
    for j in range(_RPS // 128):
        r0 = sidx * _RPS + j * 128
        pltpu.sync_copy(acc.at[pl.ds(r0, 128)],
                        hp_hbm.at[cidx, pl.ds(r0, 128)])


def _gather_pair(p_i32, q_i32, ia, ib):
    return pl.kernel(
        _sc_gather2,
        out_type=[jax.ShapeDtypeStruct((E, FP), jnp.int32)] * 2,
        mesh=_mesh(),
        scratch_types=(
            [pltpu.VMEM((_EPW,), jnp.int32)] * 2
            + [pltpu.VMEM((_BG, FP), jnp.int32)] * 8
            + [pltpu.SemaphoreType.DMA] * 16
        ),
    )(p_i32, q_i32, ia, ib)


@jax.jit
def kernel(x, edge_index, bn_w, bn_b, in1_W, in1_b, in2_W, in2_b,
           conv1_W, conv1_b, conv2_W, conv2_b, edge1_W, edge1_b,
           edge2_W, edge2_b):
    row = edge_index[0]
    col = edge_index[1]

    nblk = 5
    rb = N // nblk
    full = lambda shape: pl.BlockSpec(shape, lambda i: (0, 0))
    X, p, q = pl.pallas_call(
        _node1_body,
        out_shape=[jax.ShapeDtypeStruct((N, D), _f32)]
        + [jax.ShapeDtypeStruct((N, FP), jnp.int32)] * 2,
    )(x, bn_w.reshape(1, D), bn_b.reshape(1, D),
      in1_W, in1_b.reshape(1, HD), in2_W, in2_b.reshape(1, HD),
      conv1_W, conv1_b.reshape(1, F))

    pg, qg = _gather_pair(p, q, col, row)

    eblk = 5000
    msg = pl.pallas_call(
        _conv2_body,
        grid=(E // eblk,),
        in_specs=[
            pl.BlockSpec((eblk, FP), lambda i: (i, 0)),
            pl.BlockSpec((eblk, FP), lambda i: (i, 0)),
            full((F, HD)), full((1, HD)),
        ],
        out_specs=pl.BlockSpec((eblk, HD), lambda i: (i, 0)),
        out_shape=jax.ShapeDtypeStruct((E, HD), _f32),
    )(pg, qg, conv2_W.astype(_bf16), conv2_b.reshape(1, HD))

    hp = pl.kernel(
        _sc_scatter_add,
        out_type=jax.ShapeDtypeStruct((_NC, _NPAD, D), _f32),
        mesh=_mesh(),
        scratch_types=(
            [pltpu.VMEM((_BS,), jnp.int32)] * 2
            + [pltpu.VMEM((_BS, D), _f32)] * 2
            + [pltpu.VMEM((128, D), _f32),
               pltpu.VMEM_SHARED((_NPAD, D), _f32)]
            + [pltpu.SemaphoreType.DMA] * 4
        ),
    )(msg, col)

    r, s = pl.pallas_call(
        _node2_body,
        grid=(nblk,),
        in_specs=[
            pl.BlockSpec((1, rb, D), lambda i: (0, i, 0)),
            pl.BlockSpec((1, rb, D), lambda i: (1, i, 0)),
            pl.BlockSpec((rb, D), lambda i: (i, 0)),
            full((2 * F, F)), full((1, F)),
        ],
        out_specs=[pl.BlockSpec((rb, FP), lambda i: (i, 0))] * 2,
        out_shape=[jax.ShapeDtypeStruct((N, FP), jnp.int32)] * 2,
    )(hp, hp, X, edge1_W, edge1_b.reshape(1, F))

    rg, sg = _gather_pair(r, s, row, col)

    dblk = 5000
    out3d = pl.pallas_call(
        _edgered_body,
        grid=(E // dblk,),
        in_specs=[
            pl.BlockSpec((dblk, FP), lambda i: (i, 0)),
            pl.BlockSpec((dblk, FP), lambda i: (i, 0)),
            full((1, F)), full((1, 1)),
        ],
        out_specs=pl.BlockSpec((1, 1, dblk), lambda i: (i, 0, 0)),
        out_shape=jax.ShapeDtypeStruct((E // dblk, 1, dblk), _f32),
    )(rg, sg, edge2_W.reshape(1, F).astype(_bf16), edge2_b.reshape(1, 1))

    return out3d.reshape(E)


# speedup vs baseline: 1.0911x; 1.0189x over previous
import jax
import jax.numpy as jnp
from jax import lax
from jax.experimental import pallas as pl
from jax.experimental.pallas import tpu as pltpu
from jax.experimental.pallas import tpu_sc as plsc

N = 10000
E = 320000
D = 128
HD = 128
F = HD + D
FP = F // 2
_NC = 2
_NS = 16
_NW = _NC * _NS
_EPW = E // _NW
_BG = 80
_NBG = _EPW // _BG
_BS = 80
_NBS = _EPW // _BS
_NPAD = 10240
_RPS = _NPAD // _NS

_f32 = jnp.float32
_bf16 = jnp.bfloat16

def _mesh():
    return plsc.VectorSubcoreMesh(
        core_axis_name="c", subcore_axis_name="s",
        num_cores=_NC, num_subcores=_NS)


def _pack_rows(a_f32):
    m = a_f32.shape[0]
    return pltpu.bitcast(a_f32.astype(_bf16).reshape(2 * m, FP), jnp.int32)

def _node1_body(x_ref, bw_ref, bb_ref, w1_ref, b1_ref, w2_ref, b2_ref,
                cw_ref, cb_ref, x_out_ref, p_ref, q_ref):
    x = x_ref[...]
    mu = jnp.mean(x, axis=0, keepdims=True)
    xc = x - mu
    var = jnp.mean(xc * xc, axis=0, keepdims=True)
    xb = xc * lax.rsqrt(var + 1e-5) * bw_ref[...] + bb_ref[...]
    x_out_ref[...] = xb
    t = jnp.maximum(
        jnp.dot(xb, w1_ref[...], preferred_element_type=_f32) + b1_ref[...],
        0.0)
    h0 = jnp.tanh(
        jnp.dot(t, w2_ref[...], preferred_element_type=_f32) + b2_ref[...])
    feat = jnp.concatenate([h0, xb], axis=1)
    cw = cw_ref[...]
    wb = cw[F:]
    wa = cw[:F] - wb
    p_ref[...] = _pack_rows(
        jnp.dot(feat, wa, preferred_element_type=_f32) + cb_ref[...])
    q_ref[...] = _pack_rows(jnp.dot(feat, wb, preferred_element_type=_f32))


def _conv2_body(pg_ref, qg_ref, w_ref, b_ref, m_ref):
    pb = pltpu.bitcast(pg_ref[...], _bf16)
    qb = pltpu.bitcast(qg_ref[...], _bf16)
    t = jnp.maximum(pb + qb, 0).reshape(pg_ref.shape[0], F)
    m_ref[...] = jnp.tanh(
        jnp.dot(t, w_ref[...], preferred_element_type=_f32) + b_ref[...])


def _edgered_body(rg_ref, sg_ref, w2_ref, b2_ref, o_ref):
    rb_ = pltpu.bitcast(rg_ref[...], _bf16)
    sb_ = pltpu.bitcast(sg_ref[...], _bf16)
    t = jnp.maximum(rb_ + sb_, 0).reshape(rg_ref.shape[0], F)
    z = lax.dot_general(w2_ref[...], t, (((1,), (1,)), ((), ())),
                        preferred_element_type=_f32)
    o_ref[...] = (1.0 / (1.0 + jnp.exp(-(z + b2_ref[0, 0])))).reshape(
        1, 1, z.shape[1])


def _node2_body(h1_ref, h2_ref, xb_ref, w_ref, b_ref, r_ref, s_ref):
    feat = jnp.concatenate([h1_ref[0] + h2_ref[0], xb_ref[...]], axis=1)
    w = w_ref[...]
    r_ref[...] = _pack_rows(
        jnp.dot(feat, w[:F], preferred_element_type=_f32) + b_ref[...])
    s_ref[...] = _pack_rows(jnp.dot(feat, w[F:], preferred_element_type=_f32))


def _sc_gather2(p_hbm, q_hbm, ia_hbm, ib_hbm, pg_hbm, qg_hbm,
                iaall, iball, bp0, bp1, bp2, bp3, bq0, bq1, bq2, bq3,
                sp0, sp1, sp2, sp3, sq0, sq1, sq2, sq3,
                tp0, tp1, tp2, tp3, tq0, tq1, tq2, tq3):
    wid = lax.axis_index("s") * _NC + lax.axis_index("c")
    base = wid * _EPW
    pltpu.sync_copy(ia_hbm.at[pl.ds(base, _EPW)], iaall)
    pltpu.sync_copy(ib_hbm.at[pl.ds(base, _EPW)], iball)
    bp, bq = (bp0, bp1, bp2, bp3), (bq0, bq1, bq2, bq3)
    sp, sq = (sp0, sp1, sp2, sp3), (sq0, sq1, sq2, sq3)
    tp, tq = (tp0, tp1, tp2, tp3), (tq0, tq1, tq2, tq3)

    def gather(i, b):
        pltpu.async_copy(p_hbm.at[iaall.at[pl.ds(i * _BG, _BG)]], bp[b], sp[b])
        pltpu.async_copy(q_hbm.at[iball.at[pl.ds(i * _BG, _BG)]], bq[b], sq[b])

    def wait_gather(b):
        pltpu.make_async_copy(p_hbm.at[pl.ds(0, _BG)], bp[b], sp[b]).wait()
        pltpu.make_async_copy(q_hbm.at[pl.ds(0, _BG)], bq[b], sq[b]).wait()

    def store(i, b):
        off = base + i * _BG
        pltpu.async_copy(bp[b], pg_hbm.at[pl.ds(off, _BG)], tp[b])
        pltpu.async_copy(bq[b], qg_hbm.at[pl.ds(off, _BG)], tq[b])

    def wait_store(b):
        pltpu.make_async_copy(bp[b], pg_hbm.at[pl.ds(0, _BG)], tp[b]).wait()
        pltpu.make_async_copy(bq[b], qg_hbm.at[pl.ds(0, _BG)], tq[b]).wait()

    gather(0, 0)
    gather(1, 1)

    @pl.loop(0, (_NBG - 1) // 4)
    def _j(j):
        for b in range(4):
            i = j * 4 + b
            nxt = (b + 2) % 4
            if b < 2:
                @pl.when(j > 0)
                def _():
                    wait_store(nxt)
                gather(i + 2, nxt)
            elif b == 2:
                wait_store(nxt)
                gather(i + 2, nxt)
            else:
                wait_store(nxt)

                @pl.when(j < (_NBG - 1) // 4 - 1)
                def _():
                    gather(i + 2, nxt)
            wait_gather(b)
            store(i, b)

    wait_gather(0)
    store(_NBG - 1, 0)
    wait_store(2)
    wait_store(3)
    wait_store(0)


def _sc_scatter_add(msg_hbm, col_hbm, hp_hbm,
                    cv0, cv1, mb0, mb1, zbuf, acc, sc0, sc1, sm0, sm1):
    cidx = lax.axis_index("c")
    sidx = lax.axis_index("s")
    wid = sidx * _NC + cidx

    @pl.loop(0, 128)
    def _z(r):
        for ch in range(D // 16):
            zbuf[r, pl.ds(ch * 16, 16)] = jnp.zeros((16,), _f32)

    for j in range(_RPS // 128):
        pltpu.sync_copy(zbuf, acc.at[pl.ds(sidx * _RPS + j * 128, 128)])
    plsc.subcore_barrier()

    base = wid * _EPW
    cv, mb = (cv0, cv1), (mb0, mb1)
    sc_, sm = (sc0, sc1), (sm0, sm1)

    def issue(i, b):
        off = base + i * _BS
        pltpu.async_copy(col_hbm.at[pl.ds(off, _BS)], cv[b], sc_[b])
        pltpu.async_copy(msg_hbm.at[pl.ds(off, _BS)], mb[b], sm[b])

    def drain(b):
        pltpu.make_async_copy(col_hbm.at[pl.ds(0, _BS)], cv[b], sc_[b]).wait()
        pltpu.make_async_copy(msg_hbm.at[pl.ds(0, _BS)], mb[b], sm[b]).wait()

    issue(0, 0)

    @pl.loop(0, (_NBS - 1) // 2)
    def _j(j):
        for b in range(2):
            i = j * 2 + b
            issue(i + 1, 1 - b)
            drain(b)
            pltpu.sync_copy(mb[b], acc.at[cv[b]], add=True)

    drain(0)
    pltpu.sync_copy(mb[0], acc.at[cv[0]], add=True)

    plsc.subcore_barrier()
    for j in range(_RPS // 128):
        r0 = sidx * _RPS + j * 128
        pltpu.sync_copy(acc.at[pl.ds(r0, 128)],
                        hp_hbm.at[cidx, pl.ds(r0, 128)])


def _gather_pair(p_i32, q_i32, ia, ib):
    return pl.kernel(
        _sc_gather2,
        out_type=[jax.ShapeDtypeStruct((E, FP), jnp.int32)] * 2,
        mesh=_mesh(),
        scratch_types=(
            [pltpu.VMEM((_EPW,), jnp.int32)] * 2
            + [pltpu.VMEM((_BG, FP), jnp.int32)] * 8
            + [pltpu.SemaphoreType.DMA] * 16
        ),
    )(p_i32, q_i32, ia, ib)


@jax.jit
def kernel(x, edge_index, bn_w, bn_b, in1_W, in1_b, in2_W, in2_b,
           conv1_W, conv1_b, conv2_W, conv2_b, edge1_W, edge1_b,
           edge2_W, edge2_b):
    row = edge_index[0]
    col = edge_index[1]

    nblk = 5
    rb = N // nblk
    full = lambda shape: pl.BlockSpec(shape, lambda i: (0, 0))
    X, p, q = pl.pallas_call(
        _node1_body,
        out_shape=[jax.ShapeDtypeStruct((N, D), _f32)]
        + [jax.ShapeDtypeStruct((N, FP), jnp.int32)] * 2,
    )(x, bn_w.reshape(1, D), bn_b.reshape(1, D),
      in1_W, in1_b.reshape(1, HD), in2_W, in2_b.reshape(1, HD),
      conv1_W, conv1_b.reshape(1, F))

    pg, qg = _gather_pair(p, q, col, row)

    eblk = 8000
    msg = pl.pallas_call(
        _conv2_body,
        grid=(E // eblk,),
        in_specs=[
            pl.BlockSpec((eblk, FP), lambda i: (i, 0)),
            pl.BlockSpec((eblk, FP), lambda i: (i, 0)),
            full((F, HD)), full((1, HD)),
        ],
        out_specs=pl.BlockSpec((eblk, HD), lambda i: (i, 0)),
        out_shape=jax.ShapeDtypeStruct((E, HD), _f32),
    )(pg, qg, conv2_W.astype(_bf16), conv2_b.reshape(1, HD))

    hp = pl.kernel(
        _sc_scatter_add,
        out_type=jax.ShapeDtypeStruct((_NC, _NPAD, D), _f32),
        mesh=_mesh(),
        scratch_types=(
            [pltpu.VMEM((_BS,), jnp.int32)] * 2
            + [pltpu.VMEM((_BS, D), _f32)] * 2
            + [pltpu.VMEM((128, D), _f32),
               pltpu.VMEM_SHARED((_NPAD, D), _f32)]
            + [pltpu.SemaphoreType.DMA] * 4
        ),
    )(msg, col)

    r, s = pl.pallas_call(
        _node2_body,
        grid=(nblk,),
        in_specs=[
            pl.BlockSpec((1, rb, D), lambda i: (0, i, 0)),
            pl.BlockSpec((1, rb, D), lambda i: (1, i, 0)),
            pl.BlockSpec((rb, D), lambda i: (i, 0)),
            full((2 * F, F)), full((1, F)),
        ],
        out_specs=[pl.BlockSpec((rb, FP), lambda i: (i, 0))] * 2,
        out_shape=[jax.ShapeDtypeStruct((N, FP), jnp.int32)] * 2,
    )(hp, hp, X, edge1_W, edge1_b.reshape(1, F))

    rg, sg = _gather_pair(r, s, row, col)

    dblk = 8000
    out3d = pl.pallas_call(
        _edgered_body,
        grid=(E // dblk,),
        in_specs=[
            pl.BlockSpec((dblk, FP), lambda i: (i, 0)),
            pl.BlockSpec((dblk, FP), lambda i: (i, 0)),
            full((1, F)), full((1, 1)),
        ],
        out_specs=pl.BlockSpec((1, 1, dblk), lambda i: (i, 0, 0)),
        out_shape=jax.ShapeDtypeStruct((E // dblk, 1, dblk), _f32),
    )(rg, sg, edge2_W.reshape(1, F).astype(_bf16), edge2_b.reshape(1, 1))

    return out3d.reshape(E)
